# baseline (speedup 1.0000x reference)
import functools

import jax
import jax.numpy as jnp
from jax import lax
from jax.experimental import pallas as pl
from jax.experimental.pallas import tpu as pltpu
from jax.experimental.pallas import tpu_sc as plsc

BATCH = 16384
DIM = 16
NC, NS = 2, 16
NW = NC * NS
B_PER_W = BATCH // NW
CHUNK = 128
NCHUNK = B_PER_W // CHUNK


def _sc_gather_body(user_hbm, item_hbm, umf_hbm, imf_hbm, hu_hbm, hi_hbm,
                    umf_out, imf_out, hu_out, hi_out,
                    uidx_v, iidx_v, umf_v, imf_v, hu_v, hi_v, sem):
    wid = lax.axis_index("s") * NC + lax.axis_index("c")
    base = wid * B_PER_W
    pltpu.sync_copy(user_hbm.at[pl.ds(base, B_PER_W)], uidx_v)
    pltpu.sync_copy(item_hbm.at[pl.ds(base, B_PER_W)], iidx_v)
    copies = []
    for table, idx_v, rows_v in ((umf_hbm, uidx_v, umf_v),
                                 (imf_hbm, iidx_v, imf_v),
                                 (hu_hbm, uidx_v, hu_v),
                                 (hi_hbm, iidx_v, hi_v)):
        for j in range(NCHUNK):
            copies.append(pltpu.async_copy(
                table.at[idx_v.at[pl.ds(j * CHUNK, CHUNK)]],
                rows_v.at[pl.ds(j * CHUNK, CHUNK)], sem))
    for c in copies:
        c.wait()
    pltpu.sync_copy(umf_v, umf_out.at[pl.ds(base, B_PER_W)])
    pltpu.sync_copy(imf_v, imf_out.at[pl.ds(base, B_PER_W)])
    pltpu.sync_copy(hu_v, hu_out.at[pl.ds(base, B_PER_W)])
    pltpu.sync_copy(hi_v, hi_out.at[pl.ds(base, B_PER_W)])


@functools.cache
def _sc_gather():
    return pl.kernel(
        _sc_gather_body,
        out_type=[jax.ShapeDtypeStruct((BATCH, DIM), jnp.float32)] * 4,
        mesh=plsc.VectorSubcoreMesh(core_axis_name="c", subcore_axis_name="s",
                                    num_cores=NC, num_subcores=NS),
        scratch_types=[
            pltpu.VMEM((B_PER_W,), jnp.int32),
            pltpu.VMEM((B_PER_W,), jnp.int32),
            pltpu.VMEM((B_PER_W, DIM), jnp.float32),
            pltpu.VMEM((B_PER_W, DIM), jnp.float32),
            pltpu.VMEM((B_PER_W, DIM), jnp.float32),
            pltpu.VMEM((B_PER_W, DIM), jnp.float32),
            pltpu.SemaphoreType.DMA,
        ],
        compiler_params=pltpu.CompilerParams(use_tc_tiling_on_sc=False),
    )


BLK = 2048


def _tc_mlp_body(umf, imf, hu, hi, w1u, w1i, b1, w2, b2, wpm, wph, bp, out):
    mf = umf[...] * imf[...]
    h = jnp.dot(hu[...], w1u[...], preferred_element_type=jnp.float32)
    h += jnp.dot(hi[...], w1i[...], preferred_element_type=jnp.float32)
    h = jnp.maximum(h + b1[...], 0.0)
    h = jnp.dot(h, w2[...], preferred_element_type=jnp.float32)
    h = jnp.maximum(h + b2[...], 0.0)
    logit = (jnp.sum(mf * wpm[...], axis=1) + jnp.sum(h * wph[...], axis=1)
             + bp[0, 0])
    out[...] = 1.0 / (1.0 + jnp.exp(-logit))


def _tc_mlp(umf, imf, hu, hi, w1u, w1i, b1, w2, b2, wpm, wph, bp):
    nblk = BATCH // BLK
    row_blk = pl.BlockSpec((BLK, DIM), lambda i: (i, 0))
    full = lambda a: pl.BlockSpec(a.shape, lambda i: (0,) * a.ndim)
    return pl.pallas_call(
        _tc_mlp_body,
        grid=(nblk,),
        in_specs=[row_blk, row_blk, row_blk, row_blk,
                  full(w1u), full(w1i), full(b1), full(w2), full(b2),
                  full(wpm), full(wph), full(bp)],
        out_specs=pl.BlockSpec((BLK,), lambda i: (i,)),
        out_shape=jax.ShapeDtypeStruct((BATCH,), jnp.float32),
        compiler_params=pltpu.CompilerParams(
            dimension_semantics=("arbitrary",)),
    )(umf, imf, hu, hi, w1u, w1i, b1, w2, b2, wpm, wph, bp)


def kernel(user, item, U_mf, I_mf, U_mlp, I_mlp, W1, b1, W2, b2, Wp, bp):
    umf, imf, hu, hi = _sc_gather()(user, item, U_mf, I_mf, U_mlp, I_mlp)
    w1u = W1[:, :DIM].T
    w1i = W1[:, DIM:].T
    w2 = W2.T
    wpm = Wp[:, :DIM]
    wph = Wp[:, DIM:]
    return _tc_mlp(umf, imf, hu, hi, w1u, w1i, b1.reshape(1, -1), w2,
                   b2.reshape(1, -1), wpm, wph, bp.reshape(1, 1))


# speedup vs baseline: 4.8668x; 4.8668x over previous
import functools

import jax
import jax.numpy as jnp
from jax import lax
from jax.experimental import pallas as pl
from jax.experimental.pallas import tpu as pltpu
from jax.experimental.pallas import tpu_sc as plsc

BATCH = 16384
DIM = 16
NC, NS = 2, 16
NW = NC * NS
L = 16
NROWS = 1000000
MAIN_HI = 999936
ROWS_PER_W = 31360
CW = 1024
TAILW = 64
OUT_ROWS = BATCH + 128
VCAP = 160
SEGCAP = 160
NGRP = BATCH // L


def _sc_body(ta_u, tb_u, ta_i, tb_i, user_h, item_h, gu_out, gi_out,
             idx_u, idx_i, wpk_u, wpk_i, cha, chb, tha, thb, segc, segp,
             valbuf, posb, flidx, sema, semb, semf):
    wid = lax.axis_index("s") * NC + lax.axis_index("c")
    lane = lax.iota(jnp.int32, L)
    lo_w = wid * ROWS_PER_W
    hi_w = jnp.minimum(lo_w + ROWS_PER_W, MAIN_HI)
    is_last = wid == NW - 1
    hi_ext = jnp.where(is_last, NROWS, hi_w)
    nch = (hi_w - lo_w + CW - 1) // CW

    def zero_body(r, _):
        for j in range(8):
            valbuf[r, pl.ds(16 * j, L)] = jnp.zeros((L,), jnp.float32)
        return 0
    lax.fori_loop(0, VCAP, zero_body, 0)

    pltpu.sync_copy(user_h, idx_u)
    pltpu.sync_copy(item_h, idx_i)

    def build_body(g2, st):
        nwu, nwi = st
        parts = []
        for half in range(2):
            g = 2 * g2 + half
            posv = 16 * g + lane
            ivu = idx_u[pl.ds(16 * g, L)]
            ivi = idx_i[pl.ds(16 * g, L)]
            mu = (ivu >= lo_w) & (ivu < hi_ext)
            mi = (ivi >= lo_w) & (ivi < hi_ext)
            miu = jnp.where(mu, 1, 0)
            mii = jnp.where(mi, 1, 0)
            parts.append((posv, ivu, mu, miu, plsc.cumsum(miu),
                          ivi, mi, mii, plsc.cumsum(mii)))
        for posv, ivu, mu, miu, csu, ivi, mi, mii, csi in parts:
            plsc.store_scatter(wpk_u, [nwu + csu - miu],
                               ((ivu - lo_w) << 14) | posv, mask=mu)
            nwu = nwu + jnp.sum(miu)
            plsc.store_scatter(wpk_i, [nwi + csi - mii],
                               ((ivi - lo_w) << 14) | posv, mask=mi)
            nwi = nwi + jnp.sum(mii)
        return nwu, nwi
    n_wu, n_wi = lax.fori_loop(0, NGRP // 2, build_body, (0, 0))

    def flush(nb, out):
        for j in range(8):
            pr = posb[pl.ds(16 * j, L)]
            lp = 16 * j + lane
            pr = jnp.where(lp < nb, pr, BATCH + lp)
            flidx[0, pl.ds(16 * j, L)] = pr
        pltpu.async_copy(valbuf.at[pl.ds(0, 128)], out.at[flidx.at[0]],
                         semf).wait()
        mv = jnp.maximum(nb - 128, 0)
        posb[pl.ds(0, L)] = posb[pl.ds(128, L)]

        def mv_body(r, _):
            for j in range(8):
                valbuf[r, pl.ds(16 * j, L)] = valbuf[128 + r,
                                                     pl.ds(16 * j, L)]
            return 0
        lax.fori_loop(0, mv, mv_body, 0)
        return mv

    def extract_group(refa, refb, colv, posv, rowv, m):
        plsc.store_scatter(posb, [rowv], posv, mask=m)
        for f in range(DIM):
            fv = jnp.full((L,), f, jnp.int32)
            va = plsc.load_gather(refa, [fv, colv], mask=m)
            plsc.store_scatter(valbuf, [rowv, fv], va, mask=m)
            vb = plsc.load_gather(refb, [fv, colv], mask=m)
            plsc.store_scatter(valbuf, [rowv, fv + DIM], vb, mask=m)

    def extract_seg(refa, refb, sf, nb, out):
        def body(j, nb):
            colv = segc[pl.ds(16 * j, L)]
            posv = segp[pl.ds(16 * j, L)]
            m = lane < (sf - 16 * j)
            extract_group(refa, refb, jnp.where(m, colv, 0), posv,
                          nb + lane, m)
            nb = nb + jnp.minimum(L, sf - 16 * j)
            return lax.cond(nb >= 128, lambda n: flush(n, out),
                            lambda n: n, nb)
        return lax.fori_loop(0, (sf + L - 1) // L, body, nb)

    def process_range(refa, refb, c0, rlo, rhi, nb, n_w, wpk, out,
                      wait_fn):
        def drain(sf, nb, waited):
            waited = wait_fn(waited)
            nb = extract_seg(refa, refb, 112, nb, out)
            segc[pl.ds(0, L)] = segc[pl.ds(112, L)]
            segp[pl.ds(0, L)] = segp[pl.ds(112, L)]
            return sf - 112, nb, waited

        def g_body(g, st):
            sf, nb, waited = st
            w = wpk[pl.ds(16 * g, L)]
            rv = (w >> 14) + lo_w
            pv = w & (BATCH - 1)
            m = (lane < (n_w - 16 * g)) & (rv >= rlo) & (rv < rhi)
            mi = jnp.where(m, 1, 0)
            tgt = sf + plsc.cumsum(mi) - mi
            plsc.store_scatter(segc, [tgt], jnp.where(m, rv - c0, 0), mask=m)
            plsc.store_scatter(segp, [tgt], pv, mask=m)
            sf = sf + jnp.sum(mi)
            return lax.cond(sf >= 112, drain,
                            lambda a, b, c: (a, b, c), sf, nb, waited)

        ngw = (n_w + L - 1) // L
        sf, nb, waited = lax.fori_loop(0, ngw, g_body, (0, nb, 0))
        wait_fn(waited)
        return extract_seg(refa, refb, sf, nb, out)

    for side in range(2):
        refa, refb = ((ta_u, tb_u), (ta_i, tb_i))[side]
        wpk = (wpk_u, wpk_i)[side]
        n_w = (n_wu, n_wi)[side]
        out = (gu_out, gi_out)[side]

        def chunk_body(k, nb):
            c0 = jnp.minimum(lo_w + k * CW, hi_w - CW)
            rlo = lo_w + k * CW
            rhi = jnp.minimum(rlo + CW, hi_w)
            pltpu.sync_copy(refa.at[:, pl.ds(c0, CW)], cha)
            pltpu.sync_copy(refb.at[:, pl.ds(c0, CW)], chb)

            return process_range(cha, chb, c0, rlo, rhi, nb, n_w, wpk,
                                 out, lambda w: w)
        nb = lax.fori_loop(0, nch, chunk_body, 0)

        def tail_fn(nb):
            pltpu.sync_copy(refa.at[:, pl.ds(MAIN_HI, TAILW)], tha)
            pltpu.sync_copy(refb.at[:, pl.ds(MAIN_HI, TAILW)], thb)
            return process_range(tha, thb, MAIN_HI, MAIN_HI, NROWS,
                                 nb, n_w, wpk, out, lambda w: w)
        nb = lax.cond(is_last, tail_fn, lambda n: n, nb)

        lax.while_loop(lambda n: n > 0, lambda n: flush(n, out), nb)


@functools.cache
def _sc_gather():
    return pl.kernel(
        _sc_body,
        out_type=[jax.ShapeDtypeStruct((OUT_ROWS, 128), jnp.float32)] * 2,
        mesh=plsc.VectorSubcoreMesh(core_axis_name="c", subcore_axis_name="s",
                                    num_cores=NC, num_subcores=NS),
        scratch_types=[
            pltpu.VMEM((BATCH,), jnp.int32),
            pltpu.VMEM((BATCH,), jnp.int32),
            pltpu.VMEM((BATCH,), jnp.int32),
            pltpu.VMEM((BATCH,), jnp.int32),
            pltpu.VMEM((16, CW), jnp.float32),
            pltpu.VMEM((16, CW), jnp.float32),
            pltpu.VMEM((16, TAILW), jnp.float32),
            pltpu.VMEM((16, TAILW), jnp.float32),
            pltpu.VMEM((SEGCAP,), jnp.int32),
            pltpu.VMEM((SEGCAP,), jnp.int32),
            pltpu.VMEM((VCAP, 128), jnp.float32),
            pltpu.VMEM((VCAP,), jnp.int32),
            pltpu.VMEM((1, 128), jnp.int32),
            pltpu.SemaphoreType.DMA,
            pltpu.SemaphoreType.DMA,
            pltpu.SemaphoreType.DMA,
        ],
        compiler_params=pltpu.CompilerParams(needs_layout_passes=False),
    )


BLK = 2048


def _tc_mlp_body(gu, gi, a, b, b1, w2, b2, wpm, wph, bp, out):
    u = gu[...]
    v = gi[...]
    lmf = jnp.sum(u * v * wpm[...], axis=1)
    h = jnp.dot(u, a[...], preferred_element_type=jnp.float32)
    h += jnp.dot(v, b[...], preferred_element_type=jnp.float32)
    h = jnp.maximum(h + b1[...], 0.0)
    h = jnp.dot(h, w2[...], preferred_element_type=jnp.float32)
    h = jnp.maximum(h + b2[...], 0.0)
    logit = lmf + jnp.sum(h * wph[...], axis=1) + bp[0, 0]
    out[...] = 1.0 / (1.0 + jnp.exp(-logit))


def _tc_mlp(gu, gi, a, b, b1, w2, b2, wpm, wph, bp):
    nblk = BATCH // BLK
    row_blk = pl.BlockSpec((BLK, 128), lambda i: (i, 0))
    full = lambda x: pl.BlockSpec(x.shape, lambda i: (0,) * x.ndim)
    return pl.pallas_call(
        _tc_mlp_body,
        grid=(nblk,),
        in_specs=[row_blk, row_blk, full(a), full(b), full(b1), full(w2),
                  full(b2), full(wpm), full(wph), full(bp)],
        out_specs=pl.BlockSpec((BLK,), lambda i: (i,)),
        out_shape=jax.ShapeDtypeStruct((BATCH,), jnp.float32),
        compiler_params=pltpu.CompilerParams(
            dimension_semantics=("arbitrary",)),
    )(gu, gi, a, b, b1, w2, b2, wpm, wph, bp)


def kernel(user, item, U_mf, I_mf, U_mlp, I_mlp, W1, b1, W2, b2, Wp, bp):
    gu, gi = _sc_gather()(U_mf.T, U_mlp.T, I_mf.T, I_mlp.T, user, item)
    a = jnp.zeros((128, DIM), jnp.float32).at[DIM:2 * DIM, :].set(
        W1[:, :DIM].T)
    b = jnp.zeros((128, DIM), jnp.float32).at[DIM:2 * DIM, :].set(
        W1[:, DIM:].T)
    wpm = jnp.zeros((1, 128), jnp.float32).at[0, :DIM].set(Wp[0, :DIM])
    return _tc_mlp(gu, gi, a, b, b1.reshape(1, -1), W2.T, b2.reshape(1, -1),
                   wpm, Wp[:, DIM:], bp.reshape(1, 1))
